# baseline (speedup 1.0000x reference)
import functools

import jax
import jax.numpy as jnp
from jax.experimental import pallas as pl
from jax.experimental.pallas import tpu as pltpu

_F32 = jnp.float32
_NB_A = 4
_NB_E = 2
_NB_C = 4


def _round_up(a, m):
    return (a + m - 1) // m * m


def _fused_kernel(src_ref, dst_ref, xh_ref, ef_ref, x_ref, h_ref,
                  wt_ref, wskc_ref, wmc_ref, b4_ref,
                  out_ref, *scratch,
                  et, n_et, e_pad, unroll, cc, npad, rn, d1, d2, row_tiles):
    maccs = scratch[0:_NB_A]
    aefs = scratch[_NB_A:_NB_A + _NB_E]
    caccs = scratch[_NB_A + _NB_E:_NB_A + _NB_E + _NB_C]
    hr_ref, z_ref, hb_ref = scratch[_NB_A + _NB_E + _NB_C:]
    e = pl.program_id(0)

    @pl.when(e == 0)
    def _init():
        for m in maccs:
            m[...] = jnp.zeros_like(m)
        for a in aefs:
            a[...] = jnp.zeros_like(a)

    base = e * et

    def chunk_a(k, carry):
        b = k * unroll
        for j in range(unroll):
            li = b + j
            s = src_ref[base + li]
            d = dst_ref[base + li]
            tgt = maccs[j % _NB_A]
            tgt[d] = tgt[d] + xh_ref[s]
            tge = aefs[j % _NB_E]
            tge[d] = tge[d] + ef_ref[li]
        return carry

    jax.lax.fori_loop(0, et // unroll, chunk_a, 0)

    @pl.when(e == n_et - 1)
    def _finish():
        cin = x_ref.shape[1]
        rt = npad // row_tiles
        for t in range(row_tiles):
            r0 = t * rt
            M = maccs[0][r0:r0 + rt]
            for m in maccs[1:]:
                M = M + m[r0:r0 + rt]
            M = M.reshape(rt, d2)
            A = aefs[0][r0:r0 + rt]
            for a in aefs[1:]:
                A = A + a[r0:r0 + rt]
            A = A.reshape(rt, d2 - d1)
            T = (jnp.dot(M, wt_ref[0:d2, :], preferred_element_type=_F32)
                 + jnp.dot(A, wt_ref[d1:d2, :], preferred_element_type=_F32)
                 + jnp.dot(x_ref[r0:r0 + rt], wt_ref[d2:d2 + cin, :],
                           preferred_element_type=_F32)
                 + jnp.dot(h_ref[r0:r0 + rt], wt_ref[d2 + cin:, :],
                           preferred_element_type=_F32)
                 + b4_ref[...])
            Z = jax.nn.sigmoid(T[:, 0:cc])
            R = jax.nn.sigmoid(T[:, cc:2 * cc])
            HR = h_ref[r0:r0 + rt] * R
            hb = (T[:, 2 * cc:3 * cc] + T[:, 3 * cc:4 * cc]
                  + jnp.dot(HR, wskc_ref[...], preferred_element_type=_F32))
            z_ref[r0:r0 + rt] = Z
            hb_ref[r0:r0 + rt] = hb
            hr_ref[r0:r0 + rt] = HR.reshape(rt, 1, cc)

        for cacc in caccs:
            cacc[...] = jnp.zeros_like(cacc)

        def chunk_c(k, carry):
            b = k * unroll
            for j in range(unroll):
                gi = b + j
                s = src_ref[gi]
                d = dst_ref[gi]
                tgt = caccs[j % _NB_C]
                tgt[d] = tgt[d] + hr_ref[s]
            return carry

        jax.lax.fori_loop(0, e_pad // unroll, chunk_c, 0)

        for t in range(row_tiles):
            r0 = t * rt
            cs = caccs[0][r0:r0 + rt]
            for cacc in caccs[1:]:
                cs = cs + cacc[r0:r0 + rt]
            cs = cs.reshape(rt, cc)
            ht = jnp.tanh(hb_ref[r0:r0 + rt]
                          + jnp.dot(cs, wmc_ref[...],
                                    preferred_element_type=_F32))
            z = z_ref[r0:r0 + rt]
            out_ref[r0:r0 + rt] = z * h_ref[r0:r0 + rt] + (1.0 - z) * ht


def kernel(X, H, edge_index, edge_feature,
           wmx_x, wme_x, wsk_x, bsk_x, wmx_h, wme_h, wsk_h, bsk_h):
    N, c_in = X.shape
    C = H.shape[1]
    E, De = edge_feature.shape
    unroll = 16
    et = 512
    row_tiles = 4

    e_pad = _round_up(E, et)
    n_et = e_pad // et
    npad = _round_up(N, 2 * row_tiles * 8)
    rn = npad + 16
    dummy = rn - 1
    d1 = c_in + C
    d2 = d1 + De

    src = jnp.zeros((e_pad,), jnp.int32).at[:E].set(edge_index[0].astype(jnp.int32))
    dst = jnp.full((e_pad,), dummy, jnp.int32).at[:E].set(edge_index[1].astype(jnp.int32))

    Xf = X.astype(_F32)
    Hf = H.astype(_F32)
    XH = (jnp.zeros((rn, 1, d2), _F32)
          .at[:N, 0, :c_in].set(Xf)
          .at[:N, 0, c_in:d1].set(Hf))
    EF3 = jnp.zeros((e_pad, 1, De), _F32).at[:E, 0, :].set(edge_feature.astype(_F32))
    if npad != N:
        Xd = jnp.zeros((npad, c_in), _F32).at[:N].set(Xf)
        Hd = jnp.zeros((npad, C), _F32).at[:N].set(Hf)
    else:
        Xd, Hd = Xf, Hf

    z_xc = jnp.zeros((c_in, C), _F32)
    z_cc = jnp.zeros((C, C), _F32)
    rows_sx = jnp.concatenate([wmx_x[0], wmx_x[1], wmx_x[2], z_xc], axis=1)
    rows_sh = jnp.concatenate([wmx_h[0], wmx_h[1], z_cc, z_cc], axis=1)
    rows_se = jnp.concatenate([wme_x[0] + wme_h[0], wme_x[1] + wme_h[1],
                               wme_x[2], wme_h[2]], axis=1)
    rows_x = jnp.concatenate([wsk_x[0], wsk_x[1], wsk_x[2], z_xc], axis=1)
    rows_h = jnp.concatenate([wsk_h[0], wsk_h[1], z_cc, z_cc], axis=1)
    WT = jnp.concatenate([rows_sx, rows_sh, rows_se, rows_x, rows_h],
                         axis=0).astype(_F32)
    b4 = jnp.concatenate([bsk_x[0] + bsk_h[0], bsk_x[1] + bsk_h[1],
                          bsk_x[2], bsk_h[2]], axis=1).astype(_F32)
    wskc = wsk_h[2].astype(_F32)
    wmc = wmx_h[2].astype(_F32)

    grid_spec = pltpu.PrefetchScalarGridSpec(
        num_scalar_prefetch=2,
        grid=(n_et,),
        in_specs=[
            pl.BlockSpec((rn, 1, d2), lambda e, *_: (0, 0, 0)),
            pl.BlockSpec((et, 1, De), lambda e, *_: (e, 0, 0)),
            pl.BlockSpec((npad, c_in), lambda e, *_: (0, 0)),
            pl.BlockSpec((npad, C), lambda e, *_: (0, 0)),
            pl.BlockSpec((d2 + c_in + C, 4 * C), lambda e, *_: (0, 0)),
            pl.BlockSpec((C, C), lambda e, *_: (0, 0)),
            pl.BlockSpec((C, C), lambda e, *_: (0, 0)),
            pl.BlockSpec((1, 4 * C), lambda e, *_: (0, 0)),
        ],
        out_specs=pl.BlockSpec((npad, C), lambda e, *_: (0, 0)),
        scratch_shapes=(
            [pltpu.VMEM((rn, 1, d2), _F32) for _ in range(_NB_A)]
            + [pltpu.VMEM((rn, 1, De), _F32) for _ in range(_NB_E)]
            + [pltpu.VMEM((rn, 1, C), _F32) for _ in range(_NB_C)]
            + [
                pltpu.VMEM((npad, 1, C), _F32),
                pltpu.VMEM((npad, C), _F32),
                pltpu.VMEM((npad, C), _F32),
            ]
        ),
    )
    out = pl.pallas_call(
        functools.partial(_fused_kernel, et=et, n_et=n_et, e_pad=e_pad,
                          unroll=unroll, cc=C, npad=npad, rn=rn, d1=d1, d2=d2,
                          row_tiles=row_tiles),
        out_shape=jax.ShapeDtypeStruct((npad, C), _F32),
        grid_spec=grid_spec,
        compiler_params=pltpu.CompilerParams(
            dimension_semantics=("arbitrary",),
            vmem_limit_bytes=60 * 1024 * 1024,
        ),
    )(src, dst, XH, EF3, Xd, Hd, WT, wskc, wmc, b4)

    return out if npad == N else out[:N]


# speedup vs baseline: 1.0692x; 1.0692x over previous
import functools

import jax
import jax.numpy as jnp
from jax.experimental import pallas as pl
from jax.experimental.pallas import tpu as pltpu

_F32 = jnp.float32
_NB_A = 4
_NB_C = 4


def _round_up(a, m):
    return (a + m - 1) // m * m


def _fused_kernel(src_ref, dst_ref, xh_ref, ef_ref, x_ref, h_ref,
                  wt_ref, wskc_ref, wmc_ref, b4_ref,
                  out_ref, *scratch,
                  et, n_et, e_pad, unroll, cc, npad, rn, d2, row_tiles):
    maccs = scratch[0:_NB_A]
    caccs = scratch[_NB_A:_NB_A + _NB_C]
    hr_ref, z_ref, hb_ref = scratch[_NB_A + _NB_C:]
    e = pl.program_id(0)

    @pl.when(e == 0)
    def _init():
        for m in maccs:
            m[...] = jnp.zeros_like(m)

    base = e * et

    def chunk_a(k, carry):
        b = k * unroll
        for j in range(unroll):
            li = b + j
            s = src_ref[base + li]
            d = dst_ref[base + li]
            g = xh_ref[s]
            fe = ef_ref[li]
            tgt = maccs[j % _NB_A]
            tgt[d] = tgt[d] + jnp.concatenate([g, fe], axis=1)
        return carry

    jax.lax.fori_loop(0, et // unroll, chunk_a, 0)

    @pl.when(e == n_et - 1)
    def _finish():
        cin = x_ref.shape[1]
        rt = npad // row_tiles
        for t in range(row_tiles):
            r0 = t * rt
            M = maccs[0][r0:r0 + rt]
            for m in maccs[1:]:
                M = M + m[r0:r0 + rt]
            M = M.reshape(rt, d2)
            T = (jnp.dot(M, wt_ref[0:d2, :], preferred_element_type=_F32)
                 + jnp.dot(x_ref[r0:r0 + rt], wt_ref[d2:d2 + cin, :],
                           preferred_element_type=_F32)
                 + jnp.dot(h_ref[r0:r0 + rt], wt_ref[d2 + cin:, :],
                           preferred_element_type=_F32)
                 + b4_ref[...])
            Z = jax.nn.sigmoid(T[:, 0:cc])
            R = jax.nn.sigmoid(T[:, cc:2 * cc])
            HR = h_ref[r0:r0 + rt] * R
            hb = (T[:, 2 * cc:3 * cc] + T[:, 3 * cc:4 * cc]
                  + jnp.dot(HR, wskc_ref[...], preferred_element_type=_F32))
            z_ref[r0:r0 + rt] = Z
            hb_ref[r0:r0 + rt] = hb
            hr_ref[r0:r0 + rt] = HR.reshape(rt, 1, cc)

        for cacc in caccs:
            cacc[...] = jnp.zeros_like(cacc)

        def chunk_c(k, carry):
            b = k * unroll
            for j in range(unroll):
                gi = b + j
                s = src_ref[gi]
                d = dst_ref[gi]
                tgt = caccs[j % _NB_C]
                tgt[d] = tgt[d] + hr_ref[s]
            return carry

        jax.lax.fori_loop(0, e_pad // unroll, chunk_c, 0)

        for t in range(row_tiles):
            r0 = t * rt
            cs = caccs[0][r0:r0 + rt]
            for cacc in caccs[1:]:
                cs = cs + cacc[r0:r0 + rt]
            cs = cs.reshape(rt, cc)
            ht = jnp.tanh(hb_ref[r0:r0 + rt]
                          + jnp.dot(cs, wmc_ref[...],
                                    preferred_element_type=_F32))
            z = z_ref[r0:r0 + rt]
            out_ref[r0:r0 + rt] = z * h_ref[r0:r0 + rt] + (1.0 - z) * ht


def kernel(X, H, edge_index, edge_feature,
           wmx_x, wme_x, wsk_x, bsk_x, wmx_h, wme_h, wsk_h, bsk_h):
    N, c_in = X.shape
    C = H.shape[1]
    E, De = edge_feature.shape
    unroll = 16
    et = 1024
    row_tiles = 2

    e_pad = _round_up(E, et)
    n_et = e_pad // et
    npad = _round_up(N, 2 * row_tiles * 8)
    rn = npad + 16
    dummy = rn - 1
    d1 = c_in + C
    d2 = d1 + De

    src = jnp.zeros((e_pad,), jnp.int32).at[:E].set(edge_index[0].astype(jnp.int32))
    dst = jnp.full((e_pad,), dummy, jnp.int32).at[:E].set(edge_index[1].astype(jnp.int32))

    Xf = X.astype(_F32)
    Hf = H.astype(_F32)
    XH = (jnp.zeros((rn, 1, d1), _F32)
          .at[:N, 0, :c_in].set(Xf)
          .at[:N, 0, c_in:].set(Hf))
    EF3 = jnp.zeros((e_pad, 1, De), _F32).at[:E, 0, :].set(edge_feature.astype(_F32))
    if npad != N:
        Xd = jnp.zeros((npad, c_in), _F32).at[:N].set(Xf)
        Hd = jnp.zeros((npad, C), _F32).at[:N].set(Hf)
    else:
        Xd, Hd = Xf, Hf

    z_xc = jnp.zeros((c_in, C), _F32)
    z_cc = jnp.zeros((C, C), _F32)
    rows_sx = jnp.concatenate([wmx_x[0], wmx_x[1], wmx_x[2], z_xc], axis=1)
    rows_sh = jnp.concatenate([wmx_h[0], wmx_h[1], z_cc, z_cc], axis=1)
    rows_se = jnp.concatenate([wme_x[0] + wme_h[0], wme_x[1] + wme_h[1],
                               wme_x[2], wme_h[2]], axis=1)
    rows_x = jnp.concatenate([wsk_x[0], wsk_x[1], wsk_x[2], z_xc], axis=1)
    rows_h = jnp.concatenate([wsk_h[0], wsk_h[1], z_cc, z_cc], axis=1)
    WT = jnp.concatenate([rows_sx, rows_sh, rows_se, rows_x, rows_h],
                         axis=0).astype(_F32)
    b4 = jnp.concatenate([bsk_x[0] + bsk_h[0], bsk_x[1] + bsk_h[1],
                          bsk_x[2], bsk_h[2]], axis=1).astype(_F32)
    wskc = wsk_h[2].astype(_F32)
    wmc = wmx_h[2].astype(_F32)

    grid_spec = pltpu.PrefetchScalarGridSpec(
        num_scalar_prefetch=2,
        grid=(n_et,),
        in_specs=[
            pl.BlockSpec((rn, 1, d1), lambda e, *_: (0, 0, 0)),
            pl.BlockSpec((et, 1, De), lambda e, *_: (e, 0, 0)),
            pl.BlockSpec((npad, c_in), lambda e, *_: (0, 0)),
            pl.BlockSpec((npad, C), lambda e, *_: (0, 0)),
            pl.BlockSpec((d2 + c_in + C, 4 * C), lambda e, *_: (0, 0)),
            pl.BlockSpec((C, C), lambda e, *_: (0, 0)),
            pl.BlockSpec((C, C), lambda e, *_: (0, 0)),
            pl.BlockSpec((1, 4 * C), lambda e, *_: (0, 0)),
        ],
        out_specs=pl.BlockSpec((npad, C), lambda e, *_: (0, 0)),
        scratch_shapes=(
            [pltpu.VMEM((rn, 1, d2), _F32) for _ in range(_NB_A)]
            + [pltpu.VMEM((rn, 1, C), _F32) for _ in range(_NB_C)]
            + [
                pltpu.VMEM((npad, 1, C), _F32),
                pltpu.VMEM((npad, C), _F32),
                pltpu.VMEM((npad, C), _F32),
            ]
        ),
    )
    out = pl.pallas_call(
        functools.partial(_fused_kernel, et=et, n_et=n_et, e_pad=e_pad,
                          unroll=unroll, cc=C, npad=npad, rn=rn, d2=d2,
                          row_tiles=row_tiles),
        out_shape=jax.ShapeDtypeStruct((npad, C), _F32),
        grid_spec=grid_spec,
        compiler_params=pltpu.CompilerParams(
            dimension_semantics=("arbitrary",),
            vmem_limit_bytes=60 * 1024 * 1024,
        ),
    )(src, dst, XH, EF3, Xd, Hd, WT, wskc, wmc, b4)

    return out if npad == N else out[:N]


# speedup vs baseline: 1.0731x; 1.0036x over previous
import functools

import jax
import jax.numpy as jnp
from jax.experimental import pallas as pl
from jax.experimental.pallas import tpu as pltpu

_F32 = jnp.float32
_NB_A = 4
_NB_C = 4


def _round_up(a, m):
    return (a + m - 1) // m * m


def _fused_kernel(src_ref, dst_ref, xh_ref, ef_ref, x_ref, h_ref,
                  wt_ref, wskc_ref, wmc_ref, b4_ref,
                  out_ref, *scratch,
                  et, n_et, e_pad, unroll, cc, npad, rn, d2, row_tiles):
    maccs = scratch[0:_NB_A]
    caccs = scratch[_NB_A:_NB_A + _NB_C]
    hr_ref, z_ref, hb_ref = scratch[_NB_A + _NB_C:]
    e = pl.program_id(0)

    @pl.when(e == 0)
    def _init():
        for m in maccs:
            m[...] = jnp.zeros_like(m)

    base = e * et

    def chunk_a(k, carry):
        b = k * unroll
        for j in range(unroll):
            li = b + j
            s = src_ref[base + li]
            d = dst_ref[base + li]
            g = xh_ref[s]
            fe = ef_ref[li]
            tgt = maccs[j % _NB_A]
            tgt[d] = tgt[d] + jnp.concatenate([g, fe], axis=1)
        return carry

    jax.lax.fori_loop(0, et // unroll, chunk_a, 0)

    @pl.when(e == n_et - 1)
    def _finish():
        cin = x_ref.shape[1]
        rt = npad // row_tiles
        for t in range(row_tiles):
            r0 = t * rt
            M = maccs[0][r0:r0 + rt]
            for m in maccs[1:]:
                M = M + m[r0:r0 + rt]
            M = M.reshape(rt, d2)
            T = (jnp.dot(M, wt_ref[0:d2, :], preferred_element_type=_F32)
                 + jnp.dot(x_ref[r0:r0 + rt], wt_ref[d2:d2 + cin, :],
                           preferred_element_type=_F32)
                 + jnp.dot(h_ref[r0:r0 + rt], wt_ref[d2 + cin:, :],
                           preferred_element_type=_F32)
                 + b4_ref[...])
            Z = jax.nn.sigmoid(T[:, 0:cc])
            R = jax.nn.sigmoid(T[:, cc:2 * cc])
            HR = h_ref[r0:r0 + rt] * R
            hb = (T[:, 2 * cc:3 * cc] + T[:, 3 * cc:4 * cc]
                  + jnp.dot(HR, wskc_ref[...], preferred_element_type=_F32))
            z_ref[r0:r0 + rt] = Z
            hb_ref[r0:r0 + rt] = hb
            hr_ref[r0:r0 + rt] = HR.reshape(rt, 1, cc)

        for cacc in caccs:
            cacc[...] = jnp.zeros_like(cacc)

        def chunk_c(k, carry):
            b = k * unroll
            for j in range(unroll):
                gi = b + j
                s = src_ref[gi]
                d = dst_ref[gi]
                tgt = caccs[j % _NB_C]
                tgt[d] = tgt[d] + hr_ref[s]
            return carry

        jax.lax.fori_loop(0, e_pad // unroll, chunk_c, 0)

        for t in range(row_tiles):
            r0 = t * rt
            cs = caccs[0][r0:r0 + rt]
            for cacc in caccs[1:]:
                cs = cs + cacc[r0:r0 + rt]
            cs = cs.reshape(rt, cc)
            ht = jnp.tanh(hb_ref[r0:r0 + rt]
                          + jnp.dot(cs, wmc_ref[...],
                                    preferred_element_type=_F32))
            z = z_ref[r0:r0 + rt]
            out_ref[r0:r0 + rt] = z * h_ref[r0:r0 + rt] + (1.0 - z) * ht


def kernel(X, H, edge_index, edge_feature,
           wmx_x, wme_x, wsk_x, bsk_x, wmx_h, wme_h, wsk_h, bsk_h):
    N, c_in = X.shape
    C = H.shape[1]
    E, De = edge_feature.shape
    unroll = 16
    et = 2048
    row_tiles = 2

    e_pad = _round_up(E, et)
    n_et = e_pad // et
    npad = _round_up(N, 2 * row_tiles * 8)
    rn = npad + 16
    dummy = rn - 1
    d1 = c_in + C
    d2 = d1 + De

    src = jnp.zeros((e_pad,), jnp.int32).at[:E].set(edge_index[0].astype(jnp.int32))
    dst = jnp.full((e_pad,), dummy, jnp.int32).at[:E].set(edge_index[1].astype(jnp.int32))

    Xf = X.astype(_F32)
    Hf = H.astype(_F32)
    XH = (jnp.zeros((rn, 1, d1), _F32)
          .at[:N, 0, :c_in].set(Xf)
          .at[:N, 0, c_in:].set(Hf))
    EF3 = jnp.zeros((e_pad, 1, De), _F32).at[:E, 0, :].set(edge_feature.astype(_F32))
    if npad != N:
        Xd = jnp.zeros((npad, c_in), _F32).at[:N].set(Xf)
        Hd = jnp.zeros((npad, C), _F32).at[:N].set(Hf)
    else:
        Xd, Hd = Xf, Hf

    z_xc = jnp.zeros((c_in, C), _F32)
    z_cc = jnp.zeros((C, C), _F32)
    rows_sx = jnp.concatenate([wmx_x[0], wmx_x[1], wmx_x[2], z_xc], axis=1)
    rows_sh = jnp.concatenate([wmx_h[0], wmx_h[1], z_cc, z_cc], axis=1)
    rows_se = jnp.concatenate([wme_x[0] + wme_h[0], wme_x[1] + wme_h[1],
                               wme_x[2], wme_h[2]], axis=1)
    rows_x = jnp.concatenate([wsk_x[0], wsk_x[1], wsk_x[2], z_xc], axis=1)
    rows_h = jnp.concatenate([wsk_h[0], wsk_h[1], z_cc, z_cc], axis=1)
    WT = jnp.concatenate([rows_sx, rows_sh, rows_se, rows_x, rows_h],
                         axis=0).astype(_F32)
    b4 = jnp.concatenate([bsk_x[0] + bsk_h[0], bsk_x[1] + bsk_h[1],
                          bsk_x[2], bsk_h[2]], axis=1).astype(_F32)
    wskc = wsk_h[2].astype(_F32)
    wmc = wmx_h[2].astype(_F32)

    grid_spec = pltpu.PrefetchScalarGridSpec(
        num_scalar_prefetch=2,
        grid=(n_et,),
        in_specs=[
            pl.BlockSpec((rn, 1, d1), lambda e, *_: (0, 0, 0)),
            pl.BlockSpec((et, 1, De), lambda e, *_: (e, 0, 0)),
            pl.BlockSpec((npad, c_in), lambda e, *_: (0, 0)),
            pl.BlockSpec((npad, C), lambda e, *_: (0, 0)),
            pl.BlockSpec((d2 + c_in + C, 4 * C), lambda e, *_: (0, 0)),
            pl.BlockSpec((C, C), lambda e, *_: (0, 0)),
            pl.BlockSpec((C, C), lambda e, *_: (0, 0)),
            pl.BlockSpec((1, 4 * C), lambda e, *_: (0, 0)),
        ],
        out_specs=pl.BlockSpec((npad, C), lambda e, *_: (0, 0)),
        scratch_shapes=(
            [pltpu.VMEM((rn, 1, d2), _F32) for _ in range(_NB_A)]
            + [pltpu.VMEM((rn, 1, C), _F32) for _ in range(_NB_C)]
            + [
                pltpu.VMEM((npad, 1, C), _F32),
                pltpu.VMEM((npad, C), _F32),
                pltpu.VMEM((npad, C), _F32),
            ]
        ),
    )
    out = pl.pallas_call(
        functools.partial(_fused_kernel, et=et, n_et=n_et, e_pad=e_pad,
                          unroll=unroll, cc=C, npad=npad, rn=rn, d2=d2,
                          row_tiles=row_tiles),
        out_shape=jax.ShapeDtypeStruct((npad, C), _F32),
        grid_spec=grid_spec,
        compiler_params=pltpu.CompilerParams(
            dimension_semantics=("arbitrary",),
            vmem_limit_bytes=60 * 1024 * 1024,
        ),
    )(src, dst, XH, EF3, Xd, Hd, WT, wskc, wmc, b4)

    return out if npad == N else out[:N]
